# Initial kernel scaffold; baseline (speedup 1.0000x reference)
import jax
import jax.numpy as jnp
from jax.experimental import pallas as pl


def kernel(x, edge_index, batch, W, att_src, att_dst, bias, bn_gamma, bn_beta):
    raise NotImplementedError("write your pallas kernel here")


# speedup vs baseline: 21.6013x; 21.6013x over previous
import functools

import jax
import jax.numpy as jnp
from jax import lax
from jax.experimental import pallas as pl
from jax.experimental.pallas import tpu as pltpu
from jax.experimental.pallas import tpu_sc as plsc

N = 10000
E = 320000
D = 128
H = 4
C = 32
HC = H * C

NEG_SLOPE = 0.2
BN_EPS = 1e-5

NC = 2
NS = 16
NW = NC * NS
EPT = E // NW
EB = 2000
NB = EPT // EB
CH = 16
CPB = EB // CH
ACC_W = 144
NPS = N // NS
ZR = 125

ROWS = 1000
GRID = N // ROWS


def _leaky(v):
    return jnp.where(v >= 0, v, v * NEG_SLOPE)


def _k1_body(x_ref, w_ref, acat_ref, htab_ref, a_ref, amax_ref, macc_ref):
    i = pl.program_id(0)
    h = jnp.dot(x_ref[...], w_ref[...], preferred_element_type=jnp.float32)
    a = jnp.dot(h, acat_ref[...], preferred_element_type=jnp.float32)
    a_ref[...] = a
    htab_ref[...] = jnp.concatenate([h, a], axis=1)
    cur = jnp.max(a, axis=0, keepdims=True)

    @pl.when(i == 0)
    def _():
        macc_ref[...] = cur

    @pl.when(i > 0)
    def _():
        macc_ref[...] = jnp.maximum(macc_ref[...], cur)

    amax_ref[...] = macc_ref[...]


def _run_k1(x, w, acat):
    return pl.pallas_call(
        _k1_body,
        grid=(GRID,),
        in_specs=[
            pl.BlockSpec((ROWS, D), lambda i: (i, 0)),
            pl.BlockSpec((D, HC), lambda i: (0, 0)),
            pl.BlockSpec((HC, 16), lambda i: (0, 0)),
        ],
        out_specs=[
            pl.BlockSpec((ROWS, ACC_W), lambda i: (i, 0)),
            pl.BlockSpec((ROWS, 16), lambda i: (i, 0)),
            pl.BlockSpec((1, 16), lambda i: (0, 0)),
        ],
        out_shape=[
            jax.ShapeDtypeStruct((N, ACC_W), jnp.float32),
            jax.ShapeDtypeStruct((N, 16), jnp.float32),
            jax.ShapeDtypeStruct((1, 16), jnp.float32),
        ],
        scratch_shapes=[pltpu.VMEM((1, 16), jnp.float32)],
    )(x, w, acat)


def _sc_body(htab_hbm, src_hbm, dst_hbm, a_hbm, amax_hbm, zrow_hbm,
             out_hbm,
             src_b, dst_b, abuf, amax_v, wtab, hbuf, rowbuf,
             sidx, didx, didxc, tmprow, acc, sem_h, sem_a):
    c = lax.axis_index("c")
    s = lax.axis_index("s")
    wid = c * NS + s
    ebase = wid * EPT

    pltpu.sync_copy(amax_hbm, amax_v)

    for z in range(NPS // ZR):
        pltpu.sync_copy(zrow_hbm, acc.at[pl.ds(s * NPS + z * ZR, ZR)])

    zero16 = jnp.zeros((CH,), jnp.float32)
    for e in range(CH):
        rowbuf[e, pl.ds(HC, 16)] = zero16

    plsc.subcore_barrier()

    amax_vec = amax_v[...]
    iota = lax.iota(jnp.int32, CH)
    amh = [
        jnp.take_along_axis(amax_vec, jnp.full((CH,), h, jnp.int32), axis=0,
                            mode="promise_in_bounds")
        for h in range(H)
    ]

    def chunk(i, carry):
        srcv = src_b[pl.ds(i * CH, CH)]
        dstv = dst_b[pl.ds(i * CH, CH)]
        sidx[...] = srcv
        didx[...] = dstv
        plsc.store_scatter(didxc, [iota, jnp.full((CH,), 0, jnp.int32)], dstv)
        cp_h = pltpu.async_copy(htab_hbm.at[sidx], hbuf, sem_h)
        cp_a = pltpu.async_copy(a_hbm.at[didx], abuf, sem_a)
        cp_h.wait()
        cp_a.wait()
        ws = []
        for h in range(H):
            av = plsc.load_gather(hbuf, [iota, jnp.full((CH,), HC + h, jnp.int32)])
            bv = plsc.load_gather(abuf, [iota, jnp.full((CH,), H + h, jnp.int32)])
            al = _leaky(av + bv)
            sh = _leaky(amh[h] + bv)
            w = jnp.exp(al - sh)
            ws.append(w)
            plsc.store_scatter(rowbuf, [iota, jnp.full((CH,), HC + h, jnp.int32)], w)
        for e in range(CH):
            efull = jnp.full((CH,), e, jnp.int32)
            wsp = [jnp.take_along_axis(ws[hh], efull, axis=0,
                                       mode="promise_in_bounds")
                   for hh in range(H)]
            for j in range(HC // 16):
                hv = hbuf[e, pl.ds(j * 16, 16)]
                rowbuf[e, pl.ds(j * 16, 16)] = hv * wsp[j // 2]
        for e in range(CH):
            pltpu.sync_copy(rowbuf.at[pl.ds(e, 1)],
                            acc.at[didxc.at[e]], add=True)
        return carry

    def block(b, carry):
        pltpu.sync_copy(src_hbm.at[pl.ds(ebase + b * EB, EB)], src_b)
        pltpu.sync_copy(dst_hbm.at[pl.ds(ebase + b * EB, EB)], dst_b)
        lax.fori_loop(0, CPB, chunk, 0)
        return carry

    lax.fori_loop(0, NB, block, 0)

    plsc.subcore_barrier()

    for z in range(NPS // ZR):
        r0 = s * NPS + z * ZR
        pltpu.sync_copy(acc.at[pl.ds(r0, ZR)], out_hbm.at[c, pl.ds(r0, ZR)])


def _run_sc(htab, src, dst, a, amax16, zrow):
    mesh = plsc.VectorSubcoreMesh(core_axis_name="c", subcore_axis_name="s")
    f = functools.partial(
        pl.kernel,
        out_type=jax.ShapeDtypeStruct((NC, N, ACC_W), jnp.float32),
        mesh=mesh,
        scratch_types=[
            pltpu.VMEM((EB,), jnp.int32),
            pltpu.VMEM((EB,), jnp.int32),
            pltpu.VMEM((CH, 16), jnp.float32),
            pltpu.VMEM((16,), jnp.float32),
            pltpu.VMEM((H * CH,), jnp.float32),
            pltpu.VMEM((CH, ACC_W), jnp.float32),
            pltpu.VMEM((CH, ACC_W), jnp.float32),
            pltpu.VMEM((CH,), jnp.int32),
            pltpu.VMEM((CH,), jnp.int32),
            pltpu.VMEM((CH, 1), jnp.int32),
            pltpu.VMEM((1, ACC_W), jnp.float32),
            pltpu.VMEM_SHARED((N, ACC_W), jnp.float32),
            pltpu.SemaphoreType.DMA,
            pltpu.SemaphoreType.DMA,
        ],
        compiler_params=pltpu.CompilerParams(use_tc_tiling_on_sc=False,
                                             needs_layout_passes=False),
    )(_sc_body)
    return f(htab, src, dst, a, amax16, zrow)


def _k2_body(p0_ref, p1_ref, htab_ref, a_ref, amax_ref, bias_ref, r_ref,
             pre_ref, stats_ref, sacc_ref):
    i = pl.program_id(0)
    p0 = p0_ref[0]
    p1 = p1_ref[0]
    msg = p0[:, :HC] + p1[:, :HC]
    wsum = p0[:, HC:HC + H] + p1[:, HC:HC + H]
    a = a_ref[...]
    asrc = a[:, 0:H]
    adst = a[:, H:2 * H]
    amax = amax_ref[...][:, 0:H]
    wself = jnp.exp(_leaky(asrc + adst) - _leaky(amax + adst))
    denom = wsum + wself
    wx = jnp.dot(wself, r_ref[...], preferred_element_type=jnp.float32)
    dx = jnp.dot(denom, r_ref[...], preferred_element_type=jnp.float32)
    pre = (msg + wx * htab_ref[...][:, :HC]) / dx + bias_ref[...]
    pre = jnp.maximum(pre, 0.0)
    pre_ref[...] = pre
    s1 = jnp.sum(pre, axis=0, keepdims=True)
    s2 = jnp.sum(pre * pre, axis=0, keepdims=True)
    blk = jnp.concatenate([s1, s2], axis=0)

    @pl.when(i == 0)
    def _():
        sacc_ref[...] = blk

    @pl.when(i > 0)
    def _():
        sacc_ref[...] = sacc_ref[...] + blk

    stats_ref[...] = sacc_ref[...]


def _run_k2(sc_out, htab, a, amax, bias, r):
    return pl.pallas_call(
        _k2_body,
        grid=(GRID,),
        in_specs=[
            pl.BlockSpec((1, ROWS, ACC_W), lambda i: (0, i, 0)),
            pl.BlockSpec((1, ROWS, ACC_W), lambda i: (1, i, 0)),
            pl.BlockSpec((ROWS, ACC_W), lambda i: (i, 0)),
            pl.BlockSpec((ROWS, 16), lambda i: (i, 0)),
            pl.BlockSpec((1, 16), lambda i: (0, 0)),
            pl.BlockSpec((1, HC), lambda i: (0, 0)),
            pl.BlockSpec((H, HC), lambda i: (0, 0)),
        ],
        out_specs=[
            pl.BlockSpec((ROWS, HC), lambda i: (i, 0)),
            pl.BlockSpec((2, HC), lambda i: (0, 0)),
        ],
        out_shape=[
            jax.ShapeDtypeStruct((N, HC), jnp.float32),
            jax.ShapeDtypeStruct((2, HC), jnp.float32),
        ],
        scratch_shapes=[pltpu.VMEM((2, HC), jnp.float32)],
    )(sc_out, sc_out, htab, a, amax, bias, r)


def _k3_body(pre_ref, stats_ref, gamma_ref, beta_ref, out_ref):
    st = stats_ref[...]
    mean = st[0:1, :] * (1.0 / N)
    var = st[1:2, :] * (1.0 / N) - mean * mean
    scale = lax.rsqrt(var + BN_EPS) * gamma_ref[...]
    out_ref[...] = (pre_ref[...] - mean) * scale + beta_ref[...]


def _run_k3(pre, stats, gamma, beta):
    return pl.pallas_call(
        _k3_body,
        grid=(GRID,),
        in_specs=[
            pl.BlockSpec((ROWS, HC), lambda i: (i, 0)),
            pl.BlockSpec((2, HC), lambda i: (0, 0)),
            pl.BlockSpec((1, HC), lambda i: (0, 0)),
            pl.BlockSpec((1, HC), lambda i: (0, 0)),
        ],
        out_specs=pl.BlockSpec((ROWS, HC), lambda i: (i, 0)),
        out_shape=jax.ShapeDtypeStruct((N, HC), jnp.float32),
    )(pre, stats, gamma, beta)


def kernel(x, edge_index, batch, W, att_src, att_dst, bias, bn_gamma, bn_beta):
    del batch

    ats = att_src.reshape(H, C).astype(jnp.float32)
    atd = att_dst.reshape(H, C).astype(jnp.float32)
    eye = jnp.eye(H, dtype=jnp.float32)
    a_src_m = (eye[:, None, :] * ats[:, :, None]).reshape(HC, H)
    a_dst_m = (eye[:, None, :] * atd[:, :, None]).reshape(HC, H)
    acat = jnp.concatenate([a_src_m, a_dst_m,
                            jnp.zeros((HC, 16 - 2 * H), jnp.float32)], axis=1)
    r = jnp.repeat(eye, C, axis=1)

    htab, a, amax = _run_k1(x.astype(jnp.float32), W.astype(jnp.float32), acat)

    src = edge_index[0]
    dst = edge_index[1]
    amax16 = amax.reshape(16)
    zrow = jnp.zeros((ZR, ACC_W), jnp.float32)

    sc_out = _run_sc(htab, src, dst, a, amax16, zrow)

    pre, stats = _run_k2(sc_out, htab, a, amax, bias.reshape(1, HC), r)
    return _run_k3(pre, stats, bn_gamma.reshape(1, HC), bn_beta.reshape(1, HC))
